# baseline (speedup 1.0000x reference)
import jax
import jax.numpy as jnp
from jax import lax
from jax.experimental import pallas as pl
from jax.experimental.pallas import tpu as pltpu
from jax.experimental.pallas import tpu_sc as plsc

_NC = 2
_NS = 16
_L = 16
_NW = _NC * _NS

_BATCH = 4096
_HIST = 200
_D = 64
_OUT = 128

_GRP = (_HIST + _L - 1) // _L
_HPAD = _GRP * _L
_RPW = _BATCH // _NW

_DNUMS = lax.GatherDimensionNumbers(
    offset_dims=(), collapsed_slice_dims=(0,), start_index_map=(0,))


def _perm(v, idx):
  return lax.gather(v, idx[:, None], _DNUMS, slice_sizes=(1,),
                    mode=lax.GatherScatterMode.PROMISE_IN_BOUNDS)


def _pool_body(ids_hbm, msk_hbm, table_hbm, out_hbm,
               ids_v, msk_v, idx_a, idx_b, rows_a, rows_b, n_a, n_b, out_v,
               sem_a, sem_b):
  wid = lax.axis_index("c") * _NS + lax.axis_index("s")
  base = wid * _RPW

  pltpu.sync_copy(ids_hbm.at[pl.ds(base, _RPW)], ids_v)
  pltpu.sync_copy(msk_hbm.at[pl.ds(base, _RPW)], msk_v)

  lane = lax.iota(jnp.int32, _L)
  zeros_i = jnp.zeros((_L,), jnp.int32)
  ones_i = jnp.full((_L,), 1, jnp.int32)
  zidx = jnp.zeros((_L,), jnp.int32)

  def build_idx(r, idx_v, n_ref):
    nvec = zeros_i
    for g in range(_GRP):
      if (g + 1) * _L <= _HIST:
        ids_g = ids_v_ref[r, pl.ds(g * _L, _L)]
        m_g = msk_v_ref[r, pl.ds(g * _L, _L)] != 0
      else:
        ids_g = ids_v_ref[r, pl.ds(_HIST - _L, _L)]
        m_g = jnp.logical_and(
            msk_v_ref[r, pl.ds(_HIST - _L, _L)] != 0,
            lane >= (g * _L - (_HIST - _L)))
      sel2 = ids_g + ids_g
      idx_v[pl.ds(g * _L, _L)] = jnp.where(m_g, sel2, sel2 + ones_i)
      nvec = nvec + jnp.where(m_g, ones_i, zeros_i)
    for s in (1, 2, 4, 8):
      nvec = nvec + _perm(nvec, jnp.bitwise_xor(lane, s))
    n_ref[pl.ds(0, _L)] = nvec

  ids_v_ref = ids_v
  msk_v_ref = msk_v

  def start_gather(idx_v, rows_v, sem):
    return pltpu.async_copy(table_hbm.at[idx_v], rows_v, sem)

  def finish_row(r, n_ref, idx_v, rows_v, sem):
    pltpu.make_async_copy(table_hbm.at[idx_v], rows_v, sem).wait()
    nvec = n_ref[pl.ds(0, _L)]

    def acc_body(j, acc):
      return tuple(acc[d] + rows_v[j, pl.ds(d * _L, _L)] for d in range(4))

    zero = jnp.zeros((_L,), jnp.float32)
    acc = lax.fori_loop(0, _HPAD, acc_body, (zero, zero, zero, zero),
                        unroll=8)

    nf = nvec.astype(jnp.float32)
    inv = 1.0 / jnp.maximum(nf, 1.0)
    for d in range(4):
      out_v[r, pl.ds(d * _L, _L)] = acc[d] * inv

  build_idx(0, idx_a, n_a)
  start_gather(idx_a, rows_a, sem_a)

  def pair_body(t, carry):
    r0 = 2 * t
    build_idx(r0 + 1, idx_b, n_b)
    start_gather(idx_b, rows_b, sem_b)
    finish_row(r0, n_a, idx_a, rows_a, sem_a)

    def issue_next(_):
      build_idx(r0 + 2, idx_a, n_a)
      start_gather(idx_a, rows_a, sem_a)
      return 0

    lax.cond(t < _RPW // 2 - 1, issue_next, lambda _: 0, 0)
    finish_row(r0 + 1, n_b, idx_b, rows_b, sem_b)
    return carry

  lax.fori_loop(0, _RPW // 2, pair_body, 0)
  pltpu.sync_copy(out_v, out_hbm.at[pl.ds(base, _RPW)])


@jax.jit
def _pool(input_ids, attention_mask, table):
  mesh = plsc.VectorSubcoreMesh(core_axis_name="c", subcore_axis_name="s",
                                num_cores=_NC, num_subcores=_NS)
  return pl.kernel(
      _pool_body,
      out_type=jax.ShapeDtypeStruct((_BATCH, _D), jnp.float32),
      mesh=mesh,
      compiler_params=pltpu.CompilerParams(use_tc_tiling_on_sc=False),
      scratch_types=[
          pltpu.VMEM((_RPW, _HIST), jnp.int32),
          pltpu.VMEM((_RPW, _HIST), jnp.int32),
          pltpu.VMEM((_HPAD,), jnp.int32),
          pltpu.VMEM((_HPAD,), jnp.int32),
          pltpu.VMEM((_HPAD, _D), jnp.float32),
          pltpu.VMEM((_HPAD, _D), jnp.float32),
          pltpu.VMEM((_L,), jnp.int32),
          pltpu.VMEM((_L,), jnp.int32),
          pltpu.VMEM((_RPW, _D), jnp.float32),
          pltpu.SemaphoreType.DMA,
          pltpu.SemaphoreType.DMA,
      ],
  )(input_ids, attention_mask, table)


_TCHUNK = 36864


def _relayout_body(tt_ref, o_ref):
  row = lax.broadcasted_iota(jnp.int32, (_D, 2 * _D), 0)
  col = lax.broadcasted_iota(jnp.int32, (_D, 2 * _D), 1)
  eyel = (row == col).astype(jnp.float32)
  o_ref[...] = lax.dot_general(
      tt_ref[...], eyel, (((0,), (0,)), ((), ())),
      preferred_element_type=jnp.float32)


@jax.jit
def _relayout(table):
  tt = table.T
  vocab = table.shape[0]
  grid = (vocab + _TCHUNK - 1) // _TCHUNK
  return pl.pallas_call(
      _relayout_body,
      grid=(grid,),
      in_specs=[pl.BlockSpec((_D, _TCHUNK), lambda g: (0, g))],
      out_specs=pl.BlockSpec((_TCHUNK, 2 * _D), lambda g: (g, 0)),
      out_shape=jax.ShapeDtypeStruct((vocab, 2 * _D), jnp.float32),
  )(tt)


def _head_body(x_ref, w_ref, b_ref, o_ref):
  o_ref[...] = (
      jnp.dot(x_ref[...], w_ref[...], preferred_element_type=jnp.float32)
      + b_ref[...][None, :]
  )


@jax.jit
def _head(pooled, W, b):
  return pl.pallas_call(
      _head_body,
      out_shape=jax.ShapeDtypeStruct((_BATCH, _OUT), jnp.float32),
  )(pooled, W, b)


def kernel(input_ids, attention_mask, table, W, b):
  padded = _relayout(table)
  view = padded.reshape(2 * table.shape[0], _D)
  pooled = _pool(input_ids, attention_mask, view)
  return _head(pooled, W, b)
